# Initial kernel scaffold; baseline (speedup 1.0000x reference)
import jax
import jax.numpy as jnp
from jax.experimental import pallas as pl


def kernel(seq, token_table, pos_table):
    raise NotImplementedError("write your pallas kernel here")


# speedup vs baseline: 1.1795x; 1.1795x over previous
import functools

import jax
import jax.numpy as jnp
from jax import lax
from jax.experimental import pallas as pl
from jax.experimental.pallas import tpu as pltpu
from jax.experimental.pallas import tpu_sc as plsc

B = 4096
L = 200
D = 32
NC = 2
NS = 16
NW = NC * NS
SEQ_PER_W = B // NW
LANES = 16
SPLIT = 104


def _body(seq_hbm, tok_hbm, pos_hbm, out_hbm, idx_v, rows_v, pos_v, sem):
    wid = lax.axis_index("s") * NC + lax.axis_index("c")
    base = wid * SEQ_PER_W

    pltpu.sync_copy(pos_hbm, pos_v)

    def per_seq(s, carry):
        g = base + s
        pltpu.sync_copy(seq_hbm.at[g], idx_v)
        c0 = pltpu.async_copy(
            tok_hbm.at[idx_v.at[pl.ds(0, SPLIT)]],
            rows_v.at[pl.ds(0, SPLIT)], sem)
        c1 = pltpu.async_copy(
            tok_hbm.at[idx_v.at[pl.ds(SPLIT, L - SPLIT)]],
            rows_v.at[pl.ds(SPLIT, L - SPLIT)], sem)
        c0.wait()
        c1.wait()

        def add_row(r, c):
            for h in range(D // LANES):
                sl = pl.ds(h * LANES, LANES)
                rows_v[r, sl] = rows_v[r, sl] + pos_v[r, sl]
            return c

        lax.fori_loop(0, L, add_row, 0)

        pltpu.sync_copy(rows_v, out_hbm.at[g])
        return carry

    lax.fori_loop(0, SEQ_PER_W, per_seq, 0)


def kernel(seq, token_table, pos_table):
    mesh = plsc.VectorSubcoreMesh(
        core_axis_name="c", subcore_axis_name="s",
        num_cores=NC, num_subcores=NS)
    k = functools.partial(
        pl.kernel,
        out_type=jax.ShapeDtypeStruct((B, L, D), jnp.float32),
        mesh=mesh,
        scratch_types=[
            pltpu.VMEM((L,), jnp.int32),
            pltpu.VMEM((L, D), jnp.float32),
            pltpu.VMEM((L, D), jnp.float32),
            pltpu.SemaphoreType.DMA,
        ],
        compiler_params=pltpu.CompilerParams(use_tc_tiling_on_sc=False),
    )(_body)
    return k(seq, token_table, pos_table)


# speedup vs baseline: 1.4430x; 1.2233x over previous
import functools

import jax
import jax.numpy as jnp
from jax import lax
from jax.experimental import pallas as pl
from jax.experimental.pallas import tpu as pltpu
from jax.experimental.pallas import tpu_sc as plsc

B = 4096
L = 200
D = 32
NC = 2
NS = 16
NW = NC * NS
SEQ_PER_W = B // NW
LANES = 16
SPLIT = 104
SPLIT2 = L - SPLIT
NBUF = 4
GROUPS = SEQ_PER_W // NBUF


def _body(seq_hbm, tok_hbm, pos_hbm, out_hbm, idx_v, pos_v, *bufs):
    rows = list(bufs[0:NBUF])
    gsem = list(bufs[NBUF:2 * NBUF])
    osem = list(bufs[2 * NBUF:3 * NBUF])
    wid = lax.axis_index("s") * NC + lax.axis_index("c")
    base = wid * SEQ_PER_W

    pltpu.sync_copy(seq_hbm.at[pl.ds(base, SEQ_PER_W)], idx_v)
    pltpu.sync_copy(pos_hbm, pos_v)

    def gfire(s, b):
        pltpu.async_copy(tok_hbm.at[idx_v.at[s, pl.ds(0, SPLIT)]],
                         rows[b].at[pl.ds(0, SPLIT)], gsem[b])
        pltpu.async_copy(tok_hbm.at[idx_v.at[s, pl.ds(SPLIT, SPLIT2)]],
                         rows[b].at[pl.ds(SPLIT, SPLIT2)], gsem[b])

    def gwait(b):
        pltpu.make_async_copy(tok_hbm.at[idx_v.at[0, pl.ds(0, SPLIT)]],
                              rows[b].at[pl.ds(0, SPLIT)], gsem[b]).wait()
        pltpu.make_async_copy(tok_hbm.at[idx_v.at[0, pl.ds(SPLIT, SPLIT2)]],
                              rows[b].at[pl.ds(SPLIT, SPLIT2)], gsem[b]).wait()

    def add_pos(b):
        rb = rows[b]

        @plsc.parallel_loop(0, L, 1, unroll=8)
        def _(r):
            for h in range(D // LANES):
                sl = pl.ds(h * LANES, LANES)
                rb[r, sl] = rb[r, sl] + pos_v[r, sl]

    for b in range(NBUF):
        gfire(b, b)

    def group(g, carry):
        for b in range(NBUF):
            gwait(b)
            add_pos(b)
            pltpu.async_copy(rows[b], out_hbm.at[base + g * NBUF + b], osem[b])
        for b in range(NBUF):
            pltpu.make_async_copy(rows[b], out_hbm.at[base], osem[b]).wait()

            @pl.when(g < GROUPS - 1)
            def _fire_next():
                gfire((g + 1) * NBUF + b, b)

        return carry

    lax.fori_loop(0, GROUPS, group, 0)


def kernel(seq, token_table, pos_table):
    mesh = plsc.VectorSubcoreMesh(
        core_axis_name="c", subcore_axis_name="s",
        num_cores=NC, num_subcores=NS)
    scratch = [
        pltpu.VMEM((SEQ_PER_W, L), jnp.int32),
        pltpu.VMEM((L, D), jnp.float32),
    ]
    scratch += [pltpu.VMEM((L, D), jnp.float32) for _ in range(NBUF)]
    scratch += [pltpu.SemaphoreType.DMA for _ in range(2 * NBUF)]
    k = functools.partial(
        pl.kernel,
        out_type=jax.ShapeDtypeStruct((B, L, D), jnp.float32),
        mesh=mesh,
        scratch_types=scratch,
        compiler_params=pltpu.CompilerParams(use_tc_tiling_on_sc=False),
    )(_body)
    return k(seq, token_table, pos_table)


# speedup vs baseline: 1.6395x; 1.1362x over previous
import functools

import jax
import jax.numpy as jnp
from jax import lax
from jax.experimental import pallas as pl
from jax.experimental.pallas import tpu as pltpu
from jax.experimental.pallas import tpu_sc as plsc

B = 4096
L = 200
D = 32
NC = 2
NS = 16
NW = NC * NS
BW = B // NW
LANES = 16
NBUF = 4
GROUPS = L // NBUF


def _body(seq_hbm, tok_hbm, pos_hbm, out_hbm, seq_v, pos_v, *bufs):
    rows = list(bufs[0:NBUF])
    trans = list(bufs[NBUF:2 * NBUF])
    gsem = list(bufs[2 * NBUF:3 * NBUF])
    osem = list(bufs[3 * NBUF:4 * NBUF])
    wid = lax.axis_index("s") * NC + lax.axis_index("c")
    b0 = wid * BW

    pltpu.sync_copy(seq_hbm.at[:, pl.ds(b0, BW)], seq_v)
    pltpu.sync_copy(pos_hbm, pos_v)

    def gfire(l, b):
        pltpu.async_copy(tok_hbm.at[seq_v.at[l]], rows[b], gsem[b])

    def gwait(b):
        pltpu.make_async_copy(tok_hbm.at[seq_v.at[0]], rows[b], gsem[b]).wait()

    iota = lax.iota(jnp.int32, LANES)

    def transpose_add(l, b):
        rb = rows[b]
        tb = trans[b]

        @plsc.parallel_loop(0, D, 1, unroll=4)
        def _(f):
            fsplat = iota * 0 + f
            psplat = plsc.load_gather(pos_v, [iota * 0 + l, fsplat])
            for g in range(BW // LANES):
                bvec = iota + (g * LANES)
                v = plsc.load_gather(rb, [bvec, fsplat]) + psplat
                tb[f // 8, f % 8, pl.ds(g * LANES, LANES)] = v

    for b in range(NBUF):
        gfire(b, b)

    def group(g, carry):
        for b in range(NBUF):
            l = g * NBUF + b
            gwait(b)
            transpose_add(l, b)
            pltpu.async_copy(trans[b], out_hbm.at[l, :, wid], osem[b])
        for b in range(NBUF):
            pltpu.make_async_copy(
                trans[b], out_hbm.at[0, :, wid], osem[b]).wait()

            @pl.when(g < GROUPS - 1)
            def _fire_next():
                gfire((g + 1) * NBUF + b, b)

        return carry

    lax.fori_loop(0, GROUPS, group, 0)


def kernel(seq, token_table, pos_table):
    mesh = plsc.VectorSubcoreMesh(
        core_axis_name="c", subcore_axis_name="s",
        num_cores=NC, num_subcores=NS)
    scratch = [
        pltpu.VMEM((L, BW), jnp.int32),
        pltpu.VMEM((L, D), jnp.float32),
    ]
    scratch += [pltpu.VMEM((BW, D), jnp.float32) for _ in range(NBUF)]
    scratch += [pltpu.VMEM((D // 8, 8, BW), jnp.float32) for _ in range(NBUF)]
    scratch += [pltpu.SemaphoreType.DMA for _ in range(2 * NBUF)]
    k = functools.partial(
        pl.kernel,
        out_type=jax.ShapeDtypeStruct((L, D // 8, NW, 8, BW), jnp.float32),
        mesh=mesh,
        scratch_types=scratch,
        compiler_params=pltpu.CompilerParams(
            use_tc_tiling_on_sc=False, needs_layout_passes=False),
    )(_body)
    seq_t = seq.T
    out6 = k(seq_t, token_table, pos_table)
    return out6.transpose(2, 4, 0, 1, 3).reshape(B, L, D)
